# Initial kernel scaffold; baseline (speedup 1.0000x reference)
import jax
import jax.numpy as jnp
from jax.experimental import pallas as pl


def kernel(x, indices, control_vectors):
    raise NotImplementedError("write your pallas kernel here")


# speedup vs baseline: 19.8094x; 19.8094x over previous
import jax
import jax.numpy as jnp
from jax.experimental import pallas as pl
from jax.experimental.pallas import tpu as pltpu

BLK = 128


def _mm_body(st_ref, en_ref, bi_ref, ei_ref, xs_ref, w_ref, o_ref):
    g = pl.program_id(0)
    start = st_ref[g]
    end = en_ref[g]
    base = bi_ref[g] * BLK
    pos = base + jax.lax.broadcasted_iota(jnp.int32, (BLK, 1), 0)
    mask = (pos >= start) & (pos < end)
    xm = jnp.where(mask, xs_ref[...], 0.0)
    contrib = jax.lax.dot_general(
        xm, w_ref[0], (((1,), (1,)), ((), ())),
        preferred_element_type=jnp.float32)

    @pl.when(start == base)
    def _init():
        o_ref[...] = contrib

    @pl.when(start != base)
    def _acc():
        o_ref[...] += contrib


def kernel(x, indices, control_vectors):
    T, H = x.shape
    E = control_vectors.shape[0]
    NB = T // BLK
    G = NB + E

    sort_idx = jnp.argsort(indices)
    se = jnp.take(indices, sort_idx, axis=0)
    pos = jnp.arange(T, dtype=jnp.int32)
    prev = jnp.concatenate([se[:1], se[:-1]])
    marker = (pos % BLK == 0) | (se != prev)
    cand = jnp.where(marker, pos, T)
    cand_sorted = jnp.sort(cand)
    starts = cand_sorted[:G].astype(jnp.int32)
    ends = cand_sorted[1:G + 1].astype(jnp.int32)
    wp = jnp.minimum(starts, T - 1)
    bids = wp // BLK
    eids = jnp.take(se, wp, axis=0)

    x_sorted = jnp.take(x, sort_idx, axis=0)

    grid_spec = pltpu.PrefetchScalarGridSpec(
        num_scalar_prefetch=4,
        grid=(G,),
        in_specs=[
            pl.BlockSpec((BLK, H), lambda g, st, en, bi, ei: (bi[g], 0)),
            pl.BlockSpec((1, H, H), lambda g, st, en, bi, ei: (ei[g], 0, 0)),
        ],
        out_specs=pl.BlockSpec((BLK, H), lambda g, st, en, bi, ei: (bi[g], 0)),
    )
    out_sorted = pl.pallas_call(
        _mm_body,
        grid_spec=grid_spec,
        out_shape=jax.ShapeDtypeStruct((T, H), jnp.float32),
        compiler_params=pltpu.CompilerParams(
            dimension_semantics=("arbitrary",)),
    )(starts, ends, bids, eids, x_sorted, control_vectors)

    inv = jnp.argsort(sort_idx)
    return jnp.take(out_sorted, inv, axis=0)
